# baseline (speedup 1.0000x reference)
import jax
import jax.numpy as jnp
from jax import lax
from jax.experimental import pallas as pl
from jax.experimental.pallas import tpu as pltpu
from jax.experimental.pallas import tpu_sc as plsc

_NROWS = 32768
_D = 1024
_B = 16
_R = 32
_NG = _D // 16


def _sc_body(x_hbm, ends_hbm, o_hbm,
             ends_v, buf0, buf1, acc_v, prt_v, shared, sem0, sem1):
    c = lax.axis_index("c")
    s = lax.axis_index("s")
    seg = c * (_B // 2) + s // 2
    half = s % 2

    pltpu.sync_copy(ends_hbm, ends_v)
    evs = ends_v[...]
    seg_start = jnp.int32(0)
    seg_end = jnp.int32(0)
    for k in range(_B):
        ek = evs[k]
        seg_end = jnp.where(seg == k, ek, seg_end)
        seg_start = jnp.where(seg == k + 1, ek, seg_start)
    n = seg_end - seg_start
    n0 = n // 2
    my_lo = seg_start + half * n0
    my_hi = jnp.where(half == 0, seg_start + n0, seg_end)
    my_n = my_hi - my_lo
    lo8 = (my_lo // 8) * 8
    nchunks = jnp.where(my_n > 0, (my_hi - lo8 + _R - 1) // _R, 0)

    neg = jnp.full((16,), -jnp.inf, jnp.float32)
    for g in range(_NG):
        acc_v[pl.ds(g * 16, 16)] = neg

    bufs = (buf0, buf1)
    sems = (sem0, sem1)

    def chunk_st(k):
        return jnp.minimum(lo8 + k * _R, _NROWS - _R)

    def issue(k, b):
        pltpu.make_async_copy(
            x_hbm.at[pl.ds(chunk_st(k), _R)], bufs[b], sems[b]).start()

    def drain(b):
        pltpu.make_async_copy(
            x_hbm.at[pl.ds(0, _R)], bufs[b], sems[b]).wait()

    def process(k, b):
        st = chunk_st(k)
        j_lo = jnp.maximum(0, my_lo - st)
        j_hi = jnp.minimum(_R, my_hi - st)
        buf = bufs[b]
        for gh in range(2):
            base_g = gh * 32
            accs = tuple(
                acc_v[pl.ds((base_g + g) * 16, 16)] for g in range(32))

            def row_body(j, a):
                return tuple(
                    jnp.maximum(a[g], buf[j, pl.ds((base_g + g) * 16, 16)])
                    for g in range(32))

            accs = lax.fori_loop(j_lo, j_hi, row_body, accs)
            for g in range(32):
                acc_v[pl.ds((base_g + g) * 16, 16)] = accs[g]

    @pl.when(nchunks > 0)
    def _prime():
        issue(0, 0)

    def pair(p, carry):
        k0 = 2 * p

        @pl.when(k0 + 1 < nchunks)
        def _():
            issue(k0 + 1, 1)

        drain(0)
        process(k0, 0)

        @pl.when(k0 + 2 < nchunks)
        def _():
            issue(k0 + 2, 0)

        @pl.when(k0 + 1 < nchunks)
        def _():
            drain(1)
            process(k0 + 1, 1)

        return carry

    lax.fori_loop(0, (nchunks + 1) // 2, pair, 0)

    pltpu.sync_copy(acc_v, shared.at[pl.ds(s * _D, _D)])
    plsc.subcore_barrier()

    @pl.when(half == 0)
    def _merge():
        pltpu.sync_copy(shared.at[pl.ds((s + 1) * _D, _D)], prt_v)
        for g in range(_NG):
            sl = pl.ds(g * 16, 16)
            acc_v[sl] = jnp.maximum(acc_v[sl], prt_v[sl])
        pltpu.sync_copy(acc_v, o_hbm.at[pl.ds(seg * _D, _D)])


def kernel(input, sizes):
    ends32 = jnp.cumsum(sizes.astype(jnp.int32))
    mesh = plsc.VectorSubcoreMesh(
        core_axis_name="c", subcore_axis_name="s",
        num_cores=2, num_subcores=16)
    f = pl.kernel(
        _sc_body,
        out_type=jax.ShapeDtypeStruct((_B * _D,), jnp.float32),
        mesh=mesh,
        scratch_types=[
            pltpu.VMEM((_B,), jnp.int32),
            pltpu.VMEM((_R, _D), jnp.float32),
            pltpu.VMEM((_R, _D), jnp.float32),
            pltpu.VMEM((_D,), jnp.float32),
            pltpu.VMEM((_D,), jnp.float32),
            pltpu.VMEM_SHARED((_B * _D,), jnp.float32),
            pltpu.SemaphoreType.DMA,
            pltpu.SemaphoreType.DMA,
        ],
    )
    return f(input, ends32).reshape(_B, _D)
